# baseline (speedup 1.0000x reference)
import functools

import jax
import jax.numpy as jnp
from jax import lax
from jax.experimental import pallas as pl
from jax.experimental.pallas import tpu as pltpu
from jax.experimental.pallas import tpu_sc as plsc

N, E, F = 10000, 320000, 128
H = 8
M1, M2 = 32, 16
TDIM = F + M1 * 3 + M2 * 5
NC, NS = 2, 16
NW = NC * NS
CW = 128

PERW = E // NW
GC = 80
K = 5
NGROUP = PERW // (GC * K)

EPT = E // NS
SB = 64
NB = EPT // SB
STAIL = EPT - NB * SB
NPT = 624
NTAIL = N - NS * NPT


def _silu(v):
    return v * jax.nn.sigmoid(v)


def _fold_body(wp0, u0, wp1, u1, wp2, u2, o0, o1, o2):
    o0[...] = jnp.dot(wp0[...], u0[...], preferred_element_type=jnp.float32)
    o1[...] = jnp.dot(wp1[...], u1[...], preferred_element_type=jnp.float32)
    o2[...] = jnp.dot(wp2[...], u2[...], preferred_element_type=jnp.float32)


def _prep_body(s, w1, b1, phi):
    phi[...] = _silu(jnp.dot(s[...], w1[...],
                             preferred_element_type=jnp.float32) + b1[...])


def _edge_body(x, rbf, sh, wr, br, af, hsum, hexp, wp0f, wp1e, wp2e, p3, p5,
               out):
    r = _silu(jnp.dot(rbf[...], wr[...],
                      preferred_element_type=jnp.float32) + br[...])
    m = x[...] * r
    g = jax.nn.sigmoid(jnp.dot(m * af[...], hsum[...],
                               preferred_element_type=jnp.float32))
    mijs = m * jnp.dot(g, hexp[...], preferred_element_type=jnp.float32)
    shv = sh[...]
    mt0 = jnp.dot(mijs, wp0f[...],
                  preferred_element_type=jnp.float32) * shv[:, 0:1]
    mt1 = (jnp.dot(mijs, wp1e[...], preferred_element_type=jnp.float32)
           * jnp.dot(shv[:, 1:4], p3[...],
                     preferred_element_type=jnp.float32))
    mt2 = (jnp.dot(mijs, wp2e[...], preferred_element_type=jnp.float32)
           * jnp.dot(shv[:, 4:9], p5[...],
                     preferred_element_type=jnp.float32))
    out[0] = mijs
    out[1] = mt0
    zpad = jnp.zeros((mijs.shape[0], 32), jnp.float32)
    out[2] = jnp.concatenate([mt1, zpad], axis=1)
    out[3] = jnp.concatenate([mt2, zpad, zpad[:, :16]], axis=1)


def _update_body(s, t, agg0, agg1, wu1, bu1, wu2, bu2, s_new, t_new):
    a = agg0[...] + agg1[...]
    pay = jnp.concatenate(
        [a[0], a[1], a[2][:, :3 * M1], a[3][:, :5 * M2]], axis=1)
    aggs = pay[:, :F]
    hid = _silu(jnp.dot(aggs, wu1[...],
                        preferred_element_type=jnp.float32) + bu1[...])
    s_new[...] = s[...] + jnp.dot(hid, wu2[...],
                                  preferred_element_type=jnp.float32) + bu2[...]
    t_new[...] = t[...] + pay[:, F:F + TDIM]


_MESH = plsc.VectorSubcoreMesh(core_axis_name="c", subcore_axis_name="s")

SW = 4
SRING = 5


def _make_gather(etot, gc, k):
    perw = etot // NW
    ngroup = perw // (gc * k)
    assert perw == ngroup * gc * k and gc <= 128 and gc % 8 == 0

    @functools.partial(
        pl.kernel,
        out_type=jax.ShapeDtypeStruct((etot, F), jnp.float32),
        mesh=_MESH,
        scratch_types=[pltpu.VMEM((perw,), jnp.int32),
                       pltpu.VMEM((2 * k, gc, F), jnp.float32),
                       pltpu.SemaphoreType.DMA, pltpu.SemaphoreType.DMA,
                       pltpu.SemaphoreType.DMA],
    )
    def _gather(phi_hbm, src_hbm, x_hbm, idx_all, bufs, gsem, wsem_a, wsem_b):
        wid = lax.axis_index("s") * NC + lax.axis_index("c")
        base = wid * perw
        pltpu.sync_copy(src_hbm.at[pl.ds(base, perw)], idx_all)

        for gi in range(ngroup):
            off0 = gi * (gc * k)
            sel = (gi % 2) * k
            wsem = wsem_a if gi % 2 == 0 else wsem_b
            if gi >= 2:
                for b in range(k):
                    pltpu.make_async_copy(
                        x_hbm.at[pl.ds(base, gc)], bufs.at[sel + b],
                        wsem).wait()
            gets = [
                pltpu.async_copy(
                    phi_hbm.at[idx_all.at[pl.ds(off0 + b * gc, gc)]],
                    bufs.at[sel + b], gsem)
                for b in range(k)
            ]
            for c in gets:
                c.wait()
            for b in range(k):
                pltpu.async_copy(
                    bufs.at[sel + b],
                    x_hbm.at[pl.ds(base + off0 + b * gc, gc)], wsem)
        for gi in (ngroup - 2, ngroup - 1):
            sel = (gi % 2) * k
            wsem = wsem_a if gi % 2 == 0 else wsem_b
            for b in range(k):
                pltpu.make_async_copy(
                    x_hbm.at[pl.ds(base, gc)], bufs.at[sel + b], wsem).wait()

    return _gather


def _make_scatter(etot):
    ept = etot // NS
    nb = ept // SB
    stail = ept - nb * SB
    assert stail % 8 == 0 and stail > 0

    @functools.partial(
        pl.kernel,
        out_type=jax.ShapeDtypeStruct((4, N, CW), jnp.float32),
        mesh=_MESH,
        scratch_types=[pltpu.VMEM_SHARED((N, CW), jnp.float32),
                       pltpu.VMEM((SRING, SB), jnp.int32),
                       pltpu.VMEM((SRING, SB, CW), jnp.float32),
                       pltpu.VMEM((stail,), jnp.int32),
                       pltpu.VMEM((stail, CW), jnp.float32),
                       pltpu.SemaphoreType.DMA, pltpu.SemaphoreType.DMA],
    )
    def _scatter(msg_hbm, dst_hbm, zero_hbm, out_hbm, acc, idxb, rows,
                 idxt, rows_t, lsem, ssem):
        core = lax.axis_index("c")
        sid = lax.axis_index("s")
        ebase = sid * ept
        nbase = sid * NPT
        for cl in range(2):
            chunk = core * 2 + cl
            src = msg_hbm.at[chunk]
            pltpu.sync_copy(zero_hbm.at[pl.ds(nbase, NPT)],
                            acc.at[pl.ds(nbase, NPT)])

            @pl.when(sid == NS - 1)
            def _():
                pltpu.sync_copy(zero_hbm.at[pl.ds(NS * NPT, NTAIL)],
                                acc.at[pl.ds(NS * NPT, NTAIL)])
            plsc.subcore_barrier()

            pltpu.async_copy(dst_hbm.at[pl.ds(ebase, SB)], idxb.at[0], lsem)
            pltpu.async_copy(src.at[pl.ds(ebase, SB)], rows.at[0], lsem)

            descs = []
            for b in range(nb):
                row = b % SRING
                pltpu.make_async_copy(dst_hbm.at[pl.ds(ebase, SB)],
                                      idxb.at[row], lsem).wait()
                pltpu.make_async_copy(src.at[pl.ds(ebase, SB)],
                                      rows.at[row], lsem).wait()
                descs.append(pltpu.async_copy(
                    rows.at[row], acc.at[idxb.at[row]], ssem, add=True))
                if b >= SW:
                    descs[b - SW].wait()
                if b + 1 < nb:
                    nrow = (b + 1) % SRING
                    noff = ebase + (b + 1) * SB
                    pltpu.async_copy(dst_hbm.at[pl.ds(noff, SB)],
                                     idxb.at[nrow], lsem)
                    pltpu.async_copy(src.at[pl.ds(noff, SB)],
                                     rows.at[nrow], lsem)
            for d in descs[max(0, nb - SW):]:
                d.wait()
            toff = ebase + nb * SB
            pltpu.sync_copy(dst_hbm.at[pl.ds(toff, stail)], idxt)
            pltpu.sync_copy(src.at[pl.ds(toff, stail)], rows_t)
            pltpu.sync_copy(rows_t, acc.at[idxt], add=True)
            plsc.subcore_barrier()
            pltpu.sync_copy(acc.at[pl.ds(nbase, NPT)],
                            out_hbm.at[chunk, pl.ds(nbase, NPT)])

            @pl.when(sid == NS - 1)
            def _():
                pltpu.sync_copy(acc.at[pl.ds(NS * NPT, NTAIL)],
                                out_hbm.at[chunk, pl.ds(NS * NPT, NTAIL)])
            plsc.subcore_barrier()

    return _scatter


EH = E // 2
_sc_gather_half = _make_gather(EH, 40, 5)
_sc_scatter_half = _make_scatter(EH)


def kernel(S, T, rbf, sh, index, W1, b1, Wr, br, A, Wp0, Wp1, Wp2,
           U0, U1, U2, Wu1, bu1, Wu2, bu2):
    f32 = jnp.float32
    src = index[0].astype(jnp.int32)
    dst = index[1].astype(jnp.int32)

    hsum = (jnp.arange(F)[:, None] // (F // H)
            == jnp.arange(H)[None, :]).astype(f32)
    hexp = hsum.T
    p3 = (jnp.arange(96)[None, :] % 3
          == jnp.arange(3)[:, None]).astype(f32)
    p5 = (jnp.arange(80)[None, :] % 5
          == jnp.arange(5)[:, None]).astype(f32)
    af = A.reshape(1, F)
    b1r, brr, bu1r, bu2r = (v.reshape(1, F) for v in (b1, br, bu1, bu2))

    full = lambda shp: pl.BlockSpec(shp, lambda: tuple(0 for _ in shp))
    wp0f, wp1f, wp2f = pl.pallas_call(
        _fold_body,
        out_shape=[jax.ShapeDtypeStruct((F, F), f32),
                   jax.ShapeDtypeStruct((F, M1), f32),
                   jax.ShapeDtypeStruct((F, M2), f32)],
        in_specs=[full((F, F)), full((F, F)), full((F, M1)),
                  full((M1, M1)), full((F, M2)), full((M2, M2))],
        out_specs=[full((F, F)), full((F, M1)), full((F, M2))],
    )(Wp0, U0, Wp1, U1, Wp2, U2)
    wp1e = jnp.repeat(wp1f, 3, axis=1)
    wp2e = jnp.repeat(wp2f, 5, axis=1)

    BN = 1000
    phi = pl.pallas_call(
        _prep_body,
        grid=(N // BN,),
        out_shape=jax.ShapeDtypeStruct((N, F), f32),
        in_specs=[pl.BlockSpec((BN, F), lambda i: (i, 0)),
                  pl.BlockSpec((F, F), lambda i: (0, 0)),
                  pl.BlockSpec((1, F), lambda i: (0, 0))],
        out_specs=pl.BlockSpec((BN, F), lambda i: (i, 0)),
    )(S, W1, b1r)

    BE = 2000

    def edge_call(xh, rbfh, shh):
        return pl.pallas_call(
            _edge_body,
            grid=(EH // BE,),
            out_shape=jax.ShapeDtypeStruct((4, EH, CW), f32),
            in_specs=[pl.BlockSpec((BE, F), lambda i: (i, 0)),
                      pl.BlockSpec((BE, 8), lambda i: (i, 0)),
                      pl.BlockSpec((BE, 9), lambda i: (i, 0)),
                      pl.BlockSpec((8, F), lambda i: (0, 0)),
                      pl.BlockSpec((1, F), lambda i: (0, 0)),
                      pl.BlockSpec((1, F), lambda i: (0, 0)),
                      pl.BlockSpec((F, H), lambda i: (0, 0)),
                      pl.BlockSpec((H, F), lambda i: (0, 0)),
                      pl.BlockSpec((F, F), lambda i: (0, 0)),
                      pl.BlockSpec((F, 96), lambda i: (0, 0)),
                      pl.BlockSpec((F, 80), lambda i: (0, 0)),
                      pl.BlockSpec((3, 96), lambda i: (0, 0)),
                      pl.BlockSpec((5, 80), lambda i: (0, 0))],
            out_specs=pl.BlockSpec((4, BE, CW), lambda i: (0, i, 0)),
            compiler_params=pltpu.CompilerParams(
                dimension_semantics=("arbitrary",)),
        )(xh, rbfh, shh, Wr, brr, af, hsum, hexp, wp0f, wp1e, wp2e, p3, p5)

    zero = jnp.zeros((N, CW), f32)
    x0 = _sc_gather_half(phi, src[:EH])
    x1 = _sc_gather_half(phi, src[EH:])
    msg0 = edge_call(x0, rbf[:EH], sh[:EH])
    agg0 = _sc_scatter_half(msg0, dst[:EH], zero)
    msg1 = edge_call(x1, rbf[EH:], sh[EH:])
    agg1 = _sc_scatter_half(msg1, dst[EH:], zero)

    BU = 1000
    s_new, t_new = pl.pallas_call(
        _update_body,
        grid=(N // BU,),
        out_shape=[jax.ShapeDtypeStruct((N, F), f32),
                   jax.ShapeDtypeStruct((N, TDIM), f32)],
        in_specs=[pl.BlockSpec((BU, F), lambda i: (i, 0)),
                  pl.BlockSpec((BU, TDIM), lambda i: (i, 0)),
                  pl.BlockSpec((4, BU, CW), lambda i: (0, i, 0)),
                  pl.BlockSpec((4, BU, CW), lambda i: (0, i, 0)),
                  pl.BlockSpec((F, F), lambda i: (0, 0)),
                  pl.BlockSpec((1, F), lambda i: (0, 0)),
                  pl.BlockSpec((F, F), lambda i: (0, 0)),
                  pl.BlockSpec((1, F), lambda i: (0, 0))],
        out_specs=[pl.BlockSpec((BU, F), lambda i: (i, 0)),
                   pl.BlockSpec((BU, TDIM), lambda i: (i, 0))],
    )(S, T, agg0, agg1, Wu1, bu1r, Wu2, bu2r)
    return (s_new, t_new)


# speedup vs baseline: 1.0057x; 1.0057x over previous
import functools

import jax
import jax.numpy as jnp
from jax import lax
from jax.experimental import pallas as pl
from jax.experimental.pallas import tpu as pltpu
from jax.experimental.pallas import tpu_sc as plsc

N, E, F = 10000, 320000, 128
H = 8
M1, M2 = 32, 16
TDIM = F + M1 * 3 + M2 * 5
NC, NS = 2, 16
NW = NC * NS
CW = 128

PERW = E // NW
GC = 80
K = 5
NGROUP = PERW // (GC * K)

EPT = E // NS
SB = 64
NB = EPT // SB
STAIL = EPT - NB * SB
NPT = 624
NTAIL = N - NS * NPT


def _silu(v):
    return v * jax.nn.sigmoid(v)


def _prep_body(s, w1, b1, wp0, u0, wp1, u1, wp2, u2, phi, o0, o1, o2):
    phi[...] = _silu(jnp.dot(s[...], w1[...],
                             preferred_element_type=jnp.float32) + b1[...])
    o0[...] = jnp.dot(wp0[...], u0[...], preferred_element_type=jnp.float32)
    o1[...] = jnp.dot(wp1[...], u1[...], preferred_element_type=jnp.float32)
    o2[...] = jnp.dot(wp2[...], u2[...], preferred_element_type=jnp.float32)


def _edge_body(x, rbf, sh, wr, br, af, hsum, hexp, wp0f, wp1e, wp2e, p3, p5,
               out):
    r = _silu(jnp.dot(rbf[...], wr[...],
                      preferred_element_type=jnp.float32) + br[...])
    m = x[...] * r
    g = jax.nn.sigmoid(jnp.dot(m * af[...], hsum[...],
                               preferred_element_type=jnp.float32))
    mijs = m * jnp.dot(g, hexp[...], preferred_element_type=jnp.float32)
    shv = sh[...]
    mt0 = jnp.dot(mijs, wp0f[...],
                  preferred_element_type=jnp.float32) * shv[:, 0:1]
    mt1 = (jnp.dot(mijs, wp1e[...], preferred_element_type=jnp.float32)
           * jnp.dot(shv[:, 1:4], p3[...],
                     preferred_element_type=jnp.float32))
    mt2 = (jnp.dot(mijs, wp2e[...], preferred_element_type=jnp.float32)
           * jnp.dot(shv[:, 4:9], p5[...],
                     preferred_element_type=jnp.float32))
    out[0] = mijs
    out[1] = mt0
    zpad = jnp.zeros((mijs.shape[0], 32), jnp.float32)
    out[2] = jnp.concatenate([mt1, zpad], axis=1)
    out[3] = jnp.concatenate([mt2, zpad, zpad[:, :16]], axis=1)


def _update_body(s, t, agg0, agg1, wu1, bu1, wu2, bu2, s_new, t_new):
    a = agg0[...] + agg1[...]
    pay = jnp.concatenate(
        [a[0], a[1], a[2][:, :3 * M1], a[3][:, :5 * M2]], axis=1)
    aggs = pay[:, :F]
    hid = _silu(jnp.dot(aggs, wu1[...],
                        preferred_element_type=jnp.float32) + bu1[...])
    s_new[...] = s[...] + jnp.dot(hid, wu2[...],
                                  preferred_element_type=jnp.float32) + bu2[...]
    t_new[...] = t[...] + pay[:, F:F + TDIM]


_MESH = plsc.VectorSubcoreMesh(core_axis_name="c", subcore_axis_name="s")

SW = 4
SRING = 5


def _make_gather(etot, gc, k):
    perw = etot // NW
    ngroup = perw // (gc * k)
    assert perw == ngroup * gc * k and gc <= 128 and gc % 8 == 0

    @functools.partial(
        pl.kernel,
        out_type=jax.ShapeDtypeStruct((etot, F), jnp.float32),
        mesh=_MESH,
        scratch_types=[pltpu.VMEM((perw,), jnp.int32),
                       pltpu.VMEM((2 * k, gc, F), jnp.float32),
                       pltpu.SemaphoreType.DMA, pltpu.SemaphoreType.DMA,
                       pltpu.SemaphoreType.DMA],
    )
    def _gather(phi_hbm, src_hbm, x_hbm, idx_all, bufs, gsem, wsem_a, wsem_b):
        wid = lax.axis_index("s") * NC + lax.axis_index("c")
        base = wid * perw
        pltpu.sync_copy(src_hbm.at[pl.ds(base, perw)], idx_all)

        for gi in range(ngroup):
            off0 = gi * (gc * k)
            sel = (gi % 2) * k
            wsem = wsem_a if gi % 2 == 0 else wsem_b
            if gi >= 2:
                for b in range(k):
                    pltpu.make_async_copy(
                        x_hbm.at[pl.ds(base, gc)], bufs.at[sel + b],
                        wsem).wait()
            gets = [
                pltpu.async_copy(
                    phi_hbm.at[idx_all.at[pl.ds(off0 + b * gc, gc)]],
                    bufs.at[sel + b], gsem)
                for b in range(k)
            ]
            for c in gets:
                c.wait()
            for b in range(k):
                pltpu.async_copy(
                    bufs.at[sel + b],
                    x_hbm.at[pl.ds(base + off0 + b * gc, gc)], wsem)
        for gi in (ngroup - 2, ngroup - 1):
            sel = (gi % 2) * k
            wsem = wsem_a if gi % 2 == 0 else wsem_b
            for b in range(k):
                pltpu.make_async_copy(
                    x_hbm.at[pl.ds(base, gc)], bufs.at[sel + b], wsem).wait()

    return _gather


def _make_scatter(etot):
    ept = etot // NS
    nb = ept // SB
    stail = ept - nb * SB
    assert stail % 8 == 0 and stail > 0

    @functools.partial(
        pl.kernel,
        out_type=jax.ShapeDtypeStruct((4, N, CW), jnp.float32),
        mesh=_MESH,
        scratch_types=[pltpu.VMEM_SHARED((N, CW), jnp.float32),
                       pltpu.VMEM((SRING, SB), jnp.int32),
                       pltpu.VMEM((SRING, SB, CW), jnp.float32),
                       pltpu.VMEM((stail,), jnp.int32),
                       pltpu.VMEM((stail, CW), jnp.float32),
                       pltpu.SemaphoreType.DMA, pltpu.SemaphoreType.DMA],
    )
    def _scatter(msg_hbm, dst_hbm, zero_hbm, out_hbm, acc, idxb, rows,
                 idxt, rows_t, lsem, ssem):
        core = lax.axis_index("c")
        sid = lax.axis_index("s")
        ebase = sid * ept
        nbase = sid * NPT
        for cl in range(2):
            chunk = core * 2 + cl
            src = msg_hbm.at[chunk]
            pltpu.sync_copy(zero_hbm.at[pl.ds(nbase, NPT)],
                            acc.at[pl.ds(nbase, NPT)])

            @pl.when(sid == NS - 1)
            def _():
                pltpu.sync_copy(zero_hbm.at[pl.ds(NS * NPT, NTAIL)],
                                acc.at[pl.ds(NS * NPT, NTAIL)])
            plsc.subcore_barrier()

            pltpu.async_copy(dst_hbm.at[pl.ds(ebase, SB)], idxb.at[0], lsem)
            pltpu.async_copy(src.at[pl.ds(ebase, SB)], rows.at[0], lsem)

            descs = []
            for b in range(nb):
                row = b % SRING
                pltpu.make_async_copy(dst_hbm.at[pl.ds(ebase, SB)],
                                      idxb.at[row], lsem).wait()
                pltpu.make_async_copy(src.at[pl.ds(ebase, SB)],
                                      rows.at[row], lsem).wait()
                descs.append(pltpu.async_copy(
                    rows.at[row], acc.at[idxb.at[row]], ssem, add=True))
                if b >= SW:
                    descs[b - SW].wait()
                if b + 1 < nb:
                    nrow = (b + 1) % SRING
                    noff = ebase + (b + 1) * SB
                    pltpu.async_copy(dst_hbm.at[pl.ds(noff, SB)],
                                     idxb.at[nrow], lsem)
                    pltpu.async_copy(src.at[pl.ds(noff, SB)],
                                     rows.at[nrow], lsem)
            for d in descs[max(0, nb - SW):]:
                d.wait()
            toff = ebase + nb * SB
            pltpu.sync_copy(dst_hbm.at[pl.ds(toff, stail)], idxt)
            pltpu.sync_copy(src.at[pl.ds(toff, stail)], rows_t)
            pltpu.sync_copy(rows_t, acc.at[idxt], add=True)
            plsc.subcore_barrier()
            pltpu.sync_copy(acc.at[pl.ds(nbase, NPT)],
                            out_hbm.at[chunk, pl.ds(nbase, NPT)])

            @pl.when(sid == NS - 1)
            def _():
                pltpu.sync_copy(acc.at[pl.ds(NS * NPT, NTAIL)],
                                out_hbm.at[chunk, pl.ds(NS * NPT, NTAIL)])
            plsc.subcore_barrier()

    return _scatter


EH = E // 2
_sc_gather_half = _make_gather(EH, 40, 5)
_sc_scatter_half = _make_scatter(EH)


def kernel(S, T, rbf, sh, index, W1, b1, Wr, br, A, Wp0, Wp1, Wp2,
           U0, U1, U2, Wu1, bu1, Wu2, bu2):
    f32 = jnp.float32
    src = index[0].astype(jnp.int32)
    dst = index[1].astype(jnp.int32)

    hsum = (jnp.arange(F)[:, None] // (F // H)
            == jnp.arange(H)[None, :]).astype(f32)
    hexp = hsum.T
    p3 = (jnp.arange(96)[None, :] % 3
          == jnp.arange(3)[:, None]).astype(f32)
    p5 = (jnp.arange(80)[None, :] % 5
          == jnp.arange(5)[:, None]).astype(f32)
    af = A.reshape(1, F)
    b1r, brr, bu1r, bu2r = (v.reshape(1, F) for v in (b1, br, bu1, bu2))

    BN = 1000
    cfull = lambda shp: pl.BlockSpec(shp, lambda i: tuple(0 for _ in shp))
    phi, wp0f, wp1f, wp2f = pl.pallas_call(
        _prep_body,
        grid=(N // BN,),
        out_shape=[jax.ShapeDtypeStruct((N, F), f32),
                   jax.ShapeDtypeStruct((F, F), f32),
                   jax.ShapeDtypeStruct((F, M1), f32),
                   jax.ShapeDtypeStruct((F, M2), f32)],
        in_specs=[pl.BlockSpec((BN, F), lambda i: (i, 0)),
                  cfull((F, F)), cfull((1, F)),
                  cfull((F, F)), cfull((F, F)), cfull((F, M1)),
                  cfull((M1, M1)), cfull((F, M2)), cfull((M2, M2))],
        out_specs=[pl.BlockSpec((BN, F), lambda i: (i, 0)),
                   cfull((F, F)), cfull((F, M1)), cfull((F, M2))],
    )(S, W1, b1r, Wp0, U0, Wp1, U1, Wp2, U2)
    wp1e = jnp.repeat(wp1f, 3, axis=1)
    wp2e = jnp.repeat(wp2f, 5, axis=1)

    BE = 3200

    def edge_call(xh, rbfh, shh):
        return pl.pallas_call(
            _edge_body,
            grid=(EH // BE,),
            out_shape=jax.ShapeDtypeStruct((4, EH, CW), f32),
            in_specs=[pl.BlockSpec((BE, F), lambda i: (i, 0)),
                      pl.BlockSpec((BE, 8), lambda i: (i, 0)),
                      pl.BlockSpec((BE, 9), lambda i: (i, 0)),
                      pl.BlockSpec((8, F), lambda i: (0, 0)),
                      pl.BlockSpec((1, F), lambda i: (0, 0)),
                      pl.BlockSpec((1, F), lambda i: (0, 0)),
                      pl.BlockSpec((F, H), lambda i: (0, 0)),
                      pl.BlockSpec((H, F), lambda i: (0, 0)),
                      pl.BlockSpec((F, F), lambda i: (0, 0)),
                      pl.BlockSpec((F, 96), lambda i: (0, 0)),
                      pl.BlockSpec((F, 80), lambda i: (0, 0)),
                      pl.BlockSpec((3, 96), lambda i: (0, 0)),
                      pl.BlockSpec((5, 80), lambda i: (0, 0))],
            out_specs=pl.BlockSpec((4, BE, CW), lambda i: (0, i, 0)),
            compiler_params=pltpu.CompilerParams(
                dimension_semantics=("arbitrary",)),
        )(xh, rbfh, shh, Wr, brr, af, hsum, hexp, wp0f, wp1e, wp2e, p3, p5)

    zero = jnp.zeros((N, CW), f32)
    x0 = _sc_gather_half(phi, src[:EH])
    x1 = _sc_gather_half(phi, src[EH:])
    msg0 = edge_call(x0, rbf[:EH], sh[:EH])
    agg0 = _sc_scatter_half(msg0, dst[:EH], zero)
    msg1 = edge_call(x1, rbf[EH:], sh[EH:])
    agg1 = _sc_scatter_half(msg1, dst[EH:], zero)

    BU = 1000
    s_new, t_new = pl.pallas_call(
        _update_body,
        grid=(N // BU,),
        out_shape=[jax.ShapeDtypeStruct((N, F), f32),
                   jax.ShapeDtypeStruct((N, TDIM), f32)],
        in_specs=[pl.BlockSpec((BU, F), lambda i: (i, 0)),
                  pl.BlockSpec((BU, TDIM), lambda i: (i, 0)),
                  pl.BlockSpec((4, BU, CW), lambda i: (0, i, 0)),
                  pl.BlockSpec((4, BU, CW), lambda i: (0, i, 0)),
                  pl.BlockSpec((F, F), lambda i: (0, 0)),
                  pl.BlockSpec((1, F), lambda i: (0, 0)),
                  pl.BlockSpec((F, F), lambda i: (0, 0)),
                  pl.BlockSpec((1, F), lambda i: (0, 0))],
        out_specs=[pl.BlockSpec((BU, F), lambda i: (i, 0)),
                   pl.BlockSpec((BU, TDIM), lambda i: (i, 0))],
    )(S, T, agg0, agg1, Wu1, bu1r, Wu2, bu2r)
    return (s_new, t_new)


# speedup vs baseline: 1.0162x; 1.0105x over previous
import functools

import jax
import jax.numpy as jnp
from jax import lax
from jax.experimental import pallas as pl
from jax.experimental.pallas import tpu as pltpu
from jax.experimental.pallas import tpu_sc as plsc

N, E, F = 10000, 320000, 128
H = 8
M1, M2 = 32, 16
TDIM = F + M1 * 3 + M2 * 5
NC, NS = 2, 16
NW = NC * NS
CW = 128

PERW = E // NW
GC = 80
K = 5
NGROUP = PERW // (GC * K)

EPT = E // NS
SB = 64
NB = EPT // SB
STAIL = EPT - NB * SB
NPT = 624
NTAIL = N - NS * NPT


def _silu(v):
    return v * jax.nn.sigmoid(v)


def _prep_body(s, w1, b1, wp0, u0, wp1, u1, wp2, u2, phi, o0, o1, o2):
    phi[...] = _silu(jnp.dot(s[...], w1[...],
                             preferred_element_type=jnp.float32) + b1[...])
    o0[...] = jnp.dot(wp0[...], u0[...], preferred_element_type=jnp.float32)
    o1[...] = jnp.dot(wp1[...], u1[...], preferred_element_type=jnp.float32)
    o2[...] = jnp.dot(wp2[...], u2[...], preferred_element_type=jnp.float32)


def _edge_body(x, rbf, sh, wr, br, af, hsum, hexp, wp0f, wp1e, wp2e, p3, p5,
               out):
    r = _silu(jnp.dot(rbf[...], wr[...],
                      preferred_element_type=jnp.float32) + br[...])
    m = x[...] * r
    g = jax.nn.sigmoid(jnp.dot(m * af[...], hsum[...],
                               preferred_element_type=jnp.float32))
    mijs = m * jnp.dot(g, hexp[...], preferred_element_type=jnp.float32)
    shv = sh[...]
    mt0 = jnp.dot(mijs, wp0f[...],
                  preferred_element_type=jnp.float32) * shv[:, 0:1]
    mt1 = (jnp.dot(mijs, wp1e[...], preferred_element_type=jnp.float32)
           * jnp.dot(shv[:, 1:4], p3[...],
                     preferred_element_type=jnp.float32))
    mt2 = (jnp.dot(mijs, wp2e[...], preferred_element_type=jnp.float32)
           * jnp.dot(shv[:, 4:9], p5[...],
                     preferred_element_type=jnp.float32))
    out[0] = mijs
    out[1] = mt0
    zpad = jnp.zeros((mijs.shape[0], 32), jnp.float32)
    out[2] = jnp.concatenate([mt1, zpad], axis=1)
    out[3] = jnp.concatenate([mt2, zpad, zpad[:, :16]], axis=1)


def _update_body(s, t, agg0, agg1, wu1, bu1, wu2, bu2, s_new, t_new):
    a = agg0[...] + agg1[...]
    pay = jnp.concatenate(
        [a[0], a[1], a[2][:, :3 * M1], a[3][:, :5 * M2]], axis=1)
    aggs = pay[:, :F]
    hid = _silu(jnp.dot(aggs, wu1[...],
                        preferred_element_type=jnp.float32) + bu1[...])
    s_new[...] = s[...] + jnp.dot(hid, wu2[...],
                                  preferred_element_type=jnp.float32) + bu2[...]
    t_new[...] = t[...] + pay[:, F:F + TDIM]


_MESH = plsc.VectorSubcoreMesh(core_axis_name="c", subcore_axis_name="s")

SGRP = 3
SRING = 4


def _make_gather(etot, gc, k):
    perw = etot // NW
    ngroup = perw // (gc * k)
    assert perw == ngroup * gc * k and gc <= 128 and gc % 8 == 0

    @functools.partial(
        pl.kernel,
        out_type=jax.ShapeDtypeStruct((etot, F), jnp.float32),
        mesh=_MESH,
        scratch_types=[pltpu.VMEM((perw,), jnp.int32),
                       pltpu.VMEM((2 * k, gc, F), jnp.float32),
                       pltpu.SemaphoreType.DMA, pltpu.SemaphoreType.DMA,
                       pltpu.SemaphoreType.DMA],
    )
    def _gather(phi_hbm, src_hbm, x_hbm, idx_all, bufs, gsem, wsem_a, wsem_b):
        wid = lax.axis_index("s") * NC + lax.axis_index("c")
        base = wid * perw
        pltpu.sync_copy(src_hbm.at[pl.ds(base, perw)], idx_all)

        for gi in range(ngroup):
            off0 = gi * (gc * k)
            sel = (gi % 2) * k
            wsem = wsem_a if gi % 2 == 0 else wsem_b
            if gi >= 2:
                for b in range(k):
                    pltpu.make_async_copy(
                        x_hbm.at[pl.ds(base, gc)], bufs.at[sel + b],
                        wsem).wait()
            gets = [
                pltpu.async_copy(
                    phi_hbm.at[idx_all.at[pl.ds(off0 + b * gc, gc)]],
                    bufs.at[sel + b], gsem)
                for b in range(k)
            ]
            for c in gets:
                c.wait()
            for b in range(k):
                pltpu.async_copy(
                    bufs.at[sel + b],
                    x_hbm.at[pl.ds(base + off0 + b * gc, gc)], wsem)
        for gi in (ngroup - 2, ngroup - 1):
            sel = (gi % 2) * k
            wsem = wsem_a if gi % 2 == 0 else wsem_b
            for b in range(k):
                pltpu.make_async_copy(
                    x_hbm.at[pl.ds(base, gc)], bufs.at[sel + b], wsem).wait()

    return _gather


def _make_scatter(etot):
    ept = etot // NS
    nb = ept // SB
    stail = ept - nb * SB
    sblk = SGRP * SRING
    assert nb % sblk == 0 and stail % 8 == 0 and stail > 0

    @functools.partial(
        pl.kernel,
        out_type=jax.ShapeDtypeStruct((4, N, CW), jnp.float32),
        mesh=_MESH,
        scratch_types=[pltpu.VMEM_SHARED((N, CW), jnp.float32),
                       pltpu.VMEM((SRING, SB), jnp.int32),
                       pltpu.VMEM((SRING, SB, CW), jnp.float32),
                       pltpu.VMEM((stail,), jnp.int32),
                       pltpu.VMEM((stail, CW), jnp.float32),
                       pltpu.SemaphoreType.DMA, pltpu.SemaphoreType.DMA],
    )
    def _scatter(msg_hbm, dst_hbm, zero_hbm, out_hbm, acc, idxb, rows,
                 idxt, rows_t, lsem, ssem):
        core = lax.axis_index("c")
        sid = lax.axis_index("s")
        ebase = sid * ept
        nbase = sid * NPT
        for cl in range(2):
            chunk = core * 2 + cl
            src = msg_hbm.at[chunk]
            pltpu.sync_copy(zero_hbm.at[pl.ds(nbase, NPT)],
                            acc.at[pl.ds(nbase, NPT)])

            @pl.when(sid == NS - 1)
            def _():
                pltpu.sync_copy(zero_hbm.at[pl.ds(NS * NPT, NTAIL)],
                                acc.at[pl.ds(NS * NPT, NTAIL)])
            plsc.subcore_barrier()

            pltpu.async_copy(dst_hbm.at[pl.ds(ebase, SB)], idxb.at[0], lsem)
            pltpu.async_copy(src.at[pl.ds(ebase, SB)], rows.at[0], lsem)

            def sblock(si, carry):
                b0 = si * sblk
                descs = []
                for j in range(sblk):
                    b = b0 + j
                    row = j % SRING
                    pltpu.make_async_copy(dst_hbm.at[pl.ds(ebase, SB)],
                                          idxb.at[row], lsem).wait()
                    pltpu.make_async_copy(src.at[pl.ds(ebase, SB)],
                                          rows.at[row], lsem).wait()
                    descs.append(pltpu.async_copy(
                        rows.at[row], acc.at[idxb.at[row]], ssem, add=True))

                    nrow = (j + 1) % SRING

                    @pl.when(b + 1 < nb)
                    def _():
                        noff = ebase + (b + 1) * SB
                        pltpu.async_copy(dst_hbm.at[pl.ds(noff, SB)],
                                         idxb.at[nrow], lsem)
                        pltpu.async_copy(src.at[pl.ds(noff, SB)],
                                         rows.at[nrow], lsem)
                    if j % SGRP == SGRP - 1:
                        for d in descs:
                            d.wait()
                        descs = []
                return carry

            lax.fori_loop(0, nb // sblk, sblock, 0)
            toff = ebase + nb * SB
            pltpu.sync_copy(dst_hbm.at[pl.ds(toff, stail)], idxt)
            pltpu.sync_copy(src.at[pl.ds(toff, stail)], rows_t)
            pltpu.sync_copy(rows_t, acc.at[idxt], add=True)
            plsc.subcore_barrier()
            pltpu.sync_copy(acc.at[pl.ds(nbase, NPT)],
                            out_hbm.at[chunk, pl.ds(nbase, NPT)])

            @pl.when(sid == NS - 1)
            def _():
                pltpu.sync_copy(acc.at[pl.ds(NS * NPT, NTAIL)],
                                out_hbm.at[chunk, pl.ds(NS * NPT, NTAIL)])
            plsc.subcore_barrier()

    return _scatter


EH = E // 2
_sc_gather_half = _make_gather(EH, 40, 5)
_sc_scatter_half = _make_scatter(EH)


def kernel(S, T, rbf, sh, index, W1, b1, Wr, br, A, Wp0, Wp1, Wp2,
           U0, U1, U2, Wu1, bu1, Wu2, bu2):
    f32 = jnp.float32
    src = index[0].astype(jnp.int32)
    dst = index[1].astype(jnp.int32)

    hsum = (jnp.arange(F)[:, None] // (F // H)
            == jnp.arange(H)[None, :]).astype(f32)
    hexp = hsum.T
    p3 = (jnp.arange(96)[None, :] % 3
          == jnp.arange(3)[:, None]).astype(f32)
    p5 = (jnp.arange(80)[None, :] % 5
          == jnp.arange(5)[:, None]).astype(f32)
    af = A.reshape(1, F)
    b1r, brr, bu1r, bu2r = (v.reshape(1, F) for v in (b1, br, bu1, bu2))

    BN = 1000
    cfull = lambda shp: pl.BlockSpec(shp, lambda i: tuple(0 for _ in shp))
    phi, wp0f, wp1f, wp2f = pl.pallas_call(
        _prep_body,
        grid=(N // BN,),
        out_shape=[jax.ShapeDtypeStruct((N, F), f32),
                   jax.ShapeDtypeStruct((F, F), f32),
                   jax.ShapeDtypeStruct((F, M1), f32),
                   jax.ShapeDtypeStruct((F, M2), f32)],
        in_specs=[pl.BlockSpec((BN, F), lambda i: (i, 0)),
                  cfull((F, F)), cfull((1, F)),
                  cfull((F, F)), cfull((F, F)), cfull((F, M1)),
                  cfull((M1, M1)), cfull((F, M2)), cfull((M2, M2))],
        out_specs=[pl.BlockSpec((BN, F), lambda i: (i, 0)),
                   cfull((F, F)), cfull((F, M1)), cfull((F, M2))],
    )(S, W1, b1r, Wp0, U0, Wp1, U1, Wp2, U2)
    wp1e = jnp.repeat(wp1f, 3, axis=1)
    wp2e = jnp.repeat(wp2f, 5, axis=1)

    BE = 3200

    def edge_call(xh, rbfh, shh):
        return pl.pallas_call(
            _edge_body,
            grid=(EH // BE,),
            out_shape=jax.ShapeDtypeStruct((4, EH, CW), f32),
            in_specs=[pl.BlockSpec((BE, F), lambda i: (i, 0)),
                      pl.BlockSpec((BE, 8), lambda i: (i, 0)),
                      pl.BlockSpec((BE, 9), lambda i: (i, 0)),
                      pl.BlockSpec((8, F), lambda i: (0, 0)),
                      pl.BlockSpec((1, F), lambda i: (0, 0)),
                      pl.BlockSpec((1, F), lambda i: (0, 0)),
                      pl.BlockSpec((F, H), lambda i: (0, 0)),
                      pl.BlockSpec((H, F), lambda i: (0, 0)),
                      pl.BlockSpec((F, F), lambda i: (0, 0)),
                      pl.BlockSpec((F, 96), lambda i: (0, 0)),
                      pl.BlockSpec((F, 80), lambda i: (0, 0)),
                      pl.BlockSpec((3, 96), lambda i: (0, 0)),
                      pl.BlockSpec((5, 80), lambda i: (0, 0))],
            out_specs=pl.BlockSpec((4, BE, CW), lambda i: (0, i, 0)),
            compiler_params=pltpu.CompilerParams(
                dimension_semantics=("arbitrary",)),
        )(xh, rbfh, shh, Wr, brr, af, hsum, hexp, wp0f, wp1e, wp2e, p3, p5)

    zero = jnp.zeros((N, CW), f32)
    x0 = _sc_gather_half(phi, src[:EH])
    x1 = _sc_gather_half(phi, src[EH:])
    msg0 = edge_call(x0, rbf[:EH], sh[:EH])
    agg0 = _sc_scatter_half(msg0, dst[:EH], zero)
    msg1 = edge_call(x1, rbf[EH:], sh[EH:])
    agg1 = _sc_scatter_half(msg1, dst[EH:], zero)

    BU = 1000
    s_new, t_new = pl.pallas_call(
        _update_body,
        grid=(N // BU,),
        out_shape=[jax.ShapeDtypeStruct((N, F), f32),
                   jax.ShapeDtypeStruct((N, TDIM), f32)],
        in_specs=[pl.BlockSpec((BU, F), lambda i: (i, 0)),
                  pl.BlockSpec((BU, TDIM), lambda i: (i, 0)),
                  pl.BlockSpec((4, BU, CW), lambda i: (0, i, 0)),
                  pl.BlockSpec((4, BU, CW), lambda i: (0, i, 0)),
                  pl.BlockSpec((F, F), lambda i: (0, 0)),
                  pl.BlockSpec((1, F), lambda i: (0, 0)),
                  pl.BlockSpec((F, F), lambda i: (0, 0)),
                  pl.BlockSpec((1, F), lambda i: (0, 0))],
        out_specs=[pl.BlockSpec((BU, F), lambda i: (i, 0)),
                   pl.BlockSpec((BU, TDIM), lambda i: (i, 0))],
    )(S, T, agg0, agg1, Wu1, bu1r, Wu2, bu2r)
    return (s_new, t_new)


# speedup vs baseline: 1.0168x; 1.0006x over previous
import functools

import jax
import jax.numpy as jnp
from jax import lax
from jax.experimental import pallas as pl
from jax.experimental.pallas import tpu as pltpu
from jax.experimental.pallas import tpu_sc as plsc

N, E, F = 10000, 320000, 128
H = 8
M1, M2 = 32, 16
TDIM = F + M1 * 3 + M2 * 5
NC, NS = 2, 16
NW = NC * NS
CW = 128

SB = 64
NPT = 624
NTAIL = N - NS * NPT


def _silu(v):
    return v * jax.nn.sigmoid(v)


def _prep_body(s, w1, b1, wp0, u0, wp1, u1, wp2, u2, phi, o0, o1, o2):
    phi[...] = _silu(jnp.dot(s[...], w1[...],
                             preferred_element_type=jnp.float32) + b1[...])
    o0[...] = jnp.dot(wp0[...], u0[...], preferred_element_type=jnp.float32)
    o1[...] = jnp.dot(wp1[...], u1[...], preferred_element_type=jnp.float32)
    o2[...] = jnp.dot(wp2[...], u2[...], preferred_element_type=jnp.float32)


def _edge_body(x, rbf, sh, wr, br, af, hsum, hexp, wp0f, wp1e, wp2e, p3, p5,
               out):
    r = _silu(jnp.dot(rbf[...], wr[...],
                      preferred_element_type=jnp.float32) + br[...])
    m = x[...] * r
    g = jax.nn.sigmoid(jnp.dot(m * af[...], hsum[...],
                               preferred_element_type=jnp.float32))
    mijs = m * jnp.dot(g, hexp[...], preferred_element_type=jnp.float32)
    shv = sh[...]
    mt0 = jnp.dot(mijs, wp0f[...],
                  preferred_element_type=jnp.float32) * shv[:, 0:1]
    mt1 = (jnp.dot(mijs, wp1e[...], preferred_element_type=jnp.float32)
           * jnp.dot(shv[:, 1:4], p3[...],
                     preferred_element_type=jnp.float32))
    mt2 = (jnp.dot(mijs, wp2e[...], preferred_element_type=jnp.float32)
           * jnp.dot(shv[:, 4:9], p5[...],
                     preferred_element_type=jnp.float32))
    out[0] = mijs
    out[1] = mt0
    zpad = jnp.zeros((mijs.shape[0], 32), jnp.float32)
    out[2] = jnp.concatenate([mt1, zpad], axis=1)
    out[3] = jnp.concatenate([mt2, zpad, zpad[:, :16]], axis=1)


def _update_body(s, t, agg0, agg1, wu1, bu1, wu2, bu2, s_new, t_new):
    a = agg0[...] + agg1[...]
    pay = jnp.concatenate(
        [a[0], a[1], a[2][:, :3 * M1], a[3][:, :5 * M2]], axis=1)
    aggs = pay[:, :F]
    hid = _silu(jnp.dot(aggs, wu1[...],
                        preferred_element_type=jnp.float32) + bu1[...])
    s_new[...] = s[...] + jnp.dot(hid, wu2[...],
                                  preferred_element_type=jnp.float32) + bu2[...]
    t_new[...] = t[...] + pay[:, F:F + TDIM]


_MESH = plsc.VectorSubcoreMesh(core_axis_name="c", subcore_axis_name="s")

SGRP = 3
SRING = 4


def _make_gather(etot, gc, k):
    perw = etot // NW
    ngroup = perw // (gc * k)
    assert perw == ngroup * gc * k and gc <= 128 and gc % 8 == 0

    @functools.partial(
        pl.kernel,
        out_type=jax.ShapeDtypeStruct((etot, F), jnp.float32),
        mesh=_MESH,
        scratch_types=[pltpu.VMEM((perw,), jnp.int32),
                       pltpu.VMEM((2 * k, gc, F), jnp.float32),
                       pltpu.SemaphoreType.DMA, pltpu.SemaphoreType.DMA,
                       pltpu.SemaphoreType.DMA],
    )
    def _gather(phi_hbm, src_hbm, x_hbm, idx_all, bufs, gsem, wsem_a, wsem_b):
        wid = lax.axis_index("s") * NC + lax.axis_index("c")
        base = wid * perw
        pltpu.sync_copy(src_hbm.at[pl.ds(base, perw)], idx_all)

        for gi in range(ngroup):
            off0 = gi * (gc * k)
            sel = (gi % 2) * k
            wsem = wsem_a if gi % 2 == 0 else wsem_b
            if gi >= 2:
                for b in range(k):
                    pltpu.make_async_copy(
                        x_hbm.at[pl.ds(base, gc)], bufs.at[sel + b],
                        wsem).wait()
            gets = [
                pltpu.async_copy(
                    phi_hbm.at[idx_all.at[pl.ds(off0 + b * gc, gc)]],
                    bufs.at[sel + b], gsem)
                for b in range(k)
            ]
            for c in gets:
                c.wait()
            for b in range(k):
                pltpu.async_copy(
                    bufs.at[sel + b],
                    x_hbm.at[pl.ds(base + off0 + b * gc, gc)], wsem)
        for gi in (ngroup - 2, ngroup - 1):
            sel = (gi % 2) * k
            wsem = wsem_a if gi % 2 == 0 else wsem_b
            for b in range(k):
                pltpu.make_async_copy(
                    x_hbm.at[pl.ds(base, gc)], bufs.at[sel + b], wsem).wait()

    return _gather


def _make_scatter(etot):
    ept = etot // NS
    nb = ept // SB
    stail = ept - nb * SB
    sblk = SGRP * SRING
    assert nb % sblk == 0 and stail % 8 == 0 and stail > 0

    @functools.partial(
        pl.kernel,
        out_type=jax.ShapeDtypeStruct((4, N, CW), jnp.float32),
        mesh=_MESH,
        scratch_types=[pltpu.VMEM_SHARED((N, CW), jnp.float32),
                       pltpu.VMEM((SRING, SB), jnp.int32),
                       pltpu.VMEM((SRING, SB, CW), jnp.float32),
                       pltpu.VMEM((stail,), jnp.int32),
                       pltpu.VMEM((stail, CW), jnp.float32),
                       pltpu.SemaphoreType.DMA, pltpu.SemaphoreType.DMA],
    )
    def _scatter(msg_hbm, dst_hbm, zero_hbm, out_hbm, acc, idxb, rows,
                 idxt, rows_t, lsem, ssem):
        core = lax.axis_index("c")
        sid = lax.axis_index("s")
        ebase = sid * ept
        nbase = sid * NPT
        for cl in range(2):
            chunk = core * 2 + cl
            src = msg_hbm.at[chunk]
            pltpu.sync_copy(zero_hbm.at[pl.ds(nbase, NPT)],
                            acc.at[pl.ds(nbase, NPT)])

            @pl.when(sid == NS - 1)
            def _():
                pltpu.sync_copy(zero_hbm.at[pl.ds(NS * NPT, NTAIL)],
                                acc.at[pl.ds(NS * NPT, NTAIL)])
            plsc.subcore_barrier()

            pltpu.async_copy(dst_hbm.at[pl.ds(ebase, SB)], idxb.at[0], lsem)
            pltpu.async_copy(src.at[pl.ds(ebase, SB)], rows.at[0], lsem)

            def sblock(si, carry):
                b0 = si * sblk
                descs = []
                for j in range(sblk):
                    b = b0 + j
                    row = j % SRING
                    pltpu.make_async_copy(dst_hbm.at[pl.ds(ebase, SB)],
                                          idxb.at[row], lsem).wait()
                    pltpu.make_async_copy(src.at[pl.ds(ebase, SB)],
                                          rows.at[row], lsem).wait()
                    descs.append(pltpu.async_copy(
                        rows.at[row], acc.at[idxb.at[row]], ssem, add=True))

                    nrow = (j + 1) % SRING

                    @pl.when(b + 1 < nb)
                    def _():
                        noff = ebase + (b + 1) * SB
                        pltpu.async_copy(dst_hbm.at[pl.ds(noff, SB)],
                                         idxb.at[nrow], lsem)
                        pltpu.async_copy(src.at[pl.ds(noff, SB)],
                                         rows.at[nrow], lsem)
                    if j % SGRP == SGRP - 1:
                        for d in descs:
                            d.wait()
                        descs = []
                return carry

            lax.fori_loop(0, nb // sblk, sblock, 0)
            toff = ebase + nb * SB
            pltpu.sync_copy(dst_hbm.at[pl.ds(toff, stail)], idxt)
            pltpu.sync_copy(src.at[pl.ds(toff, stail)], rows_t)
            pltpu.sync_copy(rows_t, acc.at[idxt], add=True)
            plsc.subcore_barrier()
            pltpu.sync_copy(acc.at[pl.ds(nbase, NPT)],
                            out_hbm.at[chunk, pl.ds(nbase, NPT)])

            @pl.when(sid == NS - 1)
            def _():
                pltpu.sync_copy(acc.at[pl.ds(NS * NPT, NTAIL)],
                                out_hbm.at[chunk, pl.ds(NS * NPT, NTAIL)])
            plsc.subcore_barrier()

    return _scatter


EH = E // 2
_sc_gather_half = _make_gather(EH, 40, 5)
_sc_scatter_half = _make_scatter(EH)


def kernel(S, T, rbf, sh, index, W1, b1, Wr, br, A, Wp0, Wp1, Wp2,
           U0, U1, U2, Wu1, bu1, Wu2, bu2):
    f32 = jnp.float32
    src = index[0].astype(jnp.int32)
    dst = index[1].astype(jnp.int32)

    hsum = (jnp.arange(F)[:, None] // (F // H)
            == jnp.arange(H)[None, :]).astype(f32)
    hexp = hsum.T
    p3 = (jnp.arange(96)[None, :] % 3
          == jnp.arange(3)[:, None]).astype(f32)
    p5 = (jnp.arange(80)[None, :] % 5
          == jnp.arange(5)[:, None]).astype(f32)
    af = A.reshape(1, F)
    b1r, brr, bu1r, bu2r = (v.reshape(1, F) for v in (b1, br, bu1, bu2))

    BN = 1000
    cfull = lambda shp: pl.BlockSpec(shp, lambda i: tuple(0 for _ in shp))
    phi, wp0f, wp1f, wp2f = pl.pallas_call(
        _prep_body,
        grid=(N // BN,),
        out_shape=[jax.ShapeDtypeStruct((N, F), f32),
                   jax.ShapeDtypeStruct((F, F), f32),
                   jax.ShapeDtypeStruct((F, M1), f32),
                   jax.ShapeDtypeStruct((F, M2), f32)],
        in_specs=[pl.BlockSpec((BN, F), lambda i: (i, 0)),
                  cfull((F, F)), cfull((1, F)),
                  cfull((F, F)), cfull((F, F)), cfull((F, M1)),
                  cfull((M1, M1)), cfull((F, M2)), cfull((M2, M2))],
        out_specs=[pl.BlockSpec((BN, F), lambda i: (i, 0)),
                   cfull((F, F)), cfull((F, M1)), cfull((F, M2))],
    )(S, W1, b1r, Wp0, U0, Wp1, U1, Wp2, U2)
    wp1e = jnp.repeat(wp1f, 3, axis=1)
    wp2e = jnp.repeat(wp2f, 5, axis=1)

    BE = 3200

    def edge_call(xh, rbfh, shh):
        return pl.pallas_call(
            _edge_body,
            grid=(EH // BE,),
            out_shape=jax.ShapeDtypeStruct((4, EH, CW), f32),
            in_specs=[pl.BlockSpec((BE, F), lambda i: (i, 0)),
                      pl.BlockSpec((BE, 8), lambda i: (i, 0)),
                      pl.BlockSpec((BE, 9), lambda i: (i, 0)),
                      pl.BlockSpec((8, F), lambda i: (0, 0)),
                      pl.BlockSpec((1, F), lambda i: (0, 0)),
                      pl.BlockSpec((1, F), lambda i: (0, 0)),
                      pl.BlockSpec((F, H), lambda i: (0, 0)),
                      pl.BlockSpec((H, F), lambda i: (0, 0)),
                      pl.BlockSpec((F, F), lambda i: (0, 0)),
                      pl.BlockSpec((F, 96), lambda i: (0, 0)),
                      pl.BlockSpec((F, 80), lambda i: (0, 0)),
                      pl.BlockSpec((3, 96), lambda i: (0, 0)),
                      pl.BlockSpec((5, 80), lambda i: (0, 0))],
            out_specs=pl.BlockSpec((4, BE, CW), lambda i: (0, i, 0)),
            compiler_params=pltpu.CompilerParams(
                dimension_semantics=("arbitrary",)),
        )(xh, rbfh, shh, Wr, brr, af, hsum, hexp, wp0f, wp1e, wp2e, p3, p5)

    zero = jnp.zeros((N, CW), f32)
    x0 = _sc_gather_half(phi, src[:EH])
    x1 = _sc_gather_half(phi, src[EH:])
    msg0 = edge_call(x0, rbf[:EH], sh[:EH])
    agg0 = _sc_scatter_half(msg0, dst[:EH], zero)
    msg1 = edge_call(x1, rbf[EH:], sh[EH:])
    agg1 = _sc_scatter_half(msg1, dst[EH:], zero)

    BU = 1000
    s_new, t_new = pl.pallas_call(
        _update_body,
        grid=(N // BU,),
        out_shape=[jax.ShapeDtypeStruct((N, F), f32),
                   jax.ShapeDtypeStruct((N, TDIM), f32)],
        in_specs=[pl.BlockSpec((BU, F), lambda i: (i, 0)),
                  pl.BlockSpec((BU, TDIM), lambda i: (i, 0)),
                  pl.BlockSpec((4, BU, CW), lambda i: (0, i, 0)),
                  pl.BlockSpec((4, BU, CW), lambda i: (0, i, 0)),
                  pl.BlockSpec((F, F), lambda i: (0, 0)),
                  pl.BlockSpec((1, F), lambda i: (0, 0)),
                  pl.BlockSpec((F, F), lambda i: (0, 0)),
                  pl.BlockSpec((1, F), lambda i: (0, 0))],
        out_specs=[pl.BlockSpec((BU, F), lambda i: (i, 0)),
                   pl.BlockSpec((BU, TDIM), lambda i: (i, 0))],
    )(S, T, agg0, agg1, Wu1, bu1r, Wu2, bu2r)
    return (s_new, t_new)
